# baseline (speedup 1.0000x reference)
import functools

import jax
import jax.numpy as jnp
from jax import lax
from jax.experimental import pallas as pl
from jax.experimental.pallas import tpu as pltpu
from jax.experimental.pallas import tpu_sc as plsc

_TILES = 16
_CORES = 2
_CHUNK = 80
_NBUF = 5
_LOOK = 3
_SCAT = 2


@functools.lru_cache(maxsize=None)
def _make_segment_sum(n, e, dh):
    nw = _CORES * _TILES
    epw = e // nw
    nch = epw // _CHUNK
    rpt = (n // _TILES) // 8 * 8
    tail = _TILES * rpt
    rem = n - tail
    mesh = plsc.VectorSubcoreMesh(
        core_axis_name="c", subcore_axis_name="s",
        num_cores=_CORES, num_subcores=_TILES)

    @functools.partial(
        pl.kernel,
        out_type=jax.ShapeDtypeStruct((2, _CORES, n, dh), jnp.float32),
        mesh=mesh,
        compiler_params=pltpu.CompilerParams(use_tc_tiling_on_sc=False),
        scratch_types=[
            pltpu.VMEM((nch, _CHUNK), jnp.int32),
            pltpu.VMEM((nch, _CHUNK), jnp.int32),
            pltpu.VMEM((_NBUF, _CHUNK, dh), jnp.float32),
            pltpu.VMEM_SHARED((n, dh), jnp.float32),
            pltpu.SemaphoreType.DMA,
            pltpu.SemaphoreType.DMA,
        ],
    )
    def seg(x2_hbm, src_hbm, dst_hbm, zrows_hbm, out_hbm,
            sidx, didx, rows, acc, gsem, ssem):
        c = lax.axis_index("c")
        s = lax.axis_index("s")
        w = c * _TILES + s
        row_slice = pl.ds(s * rpt, rpt)
        tail_slice = pl.ds(tail, max(rem, 1))

        pltpu.sync_copy(dst_hbm.at[w], didx)

        def zero_acc():
            pltpu.sync_copy(zrows_hbm.at[pl.ds(0, rpt)], acc.at[row_slice])
            if rem:
                @pl.when(s == _TILES - 1)
                def _():
                    pltpu.sync_copy(zrows_hbm.at[pl.ds(0, rem)],
                                    acc.at[tail_slice])

        def accumulate(phase):
            pltpu.sync_copy(src_hbm.at[phase, w], sidx)

            def fire_gather(g, b):
                pltpu.async_copy(x2_hbm.at[sidx.at[g]], rows.at[b], gsem)

            def wait_gather(g, b):
                pltpu.make_async_copy(x2_hbm.at[sidx.at[g]], rows.at[b],
                                      gsem).wait()

            def fire_scatter(g, b):
                pltpu.async_copy(rows.at[b], acc.at[didx.at[g]], ssem,
                                 add=True)

            def wait_scatter(g, b):
                pltpu.make_async_copy(rows.at[b], acc.at[didx.at[g]],
                                      ssem).wait()

            for g in range(_LOOK):
                fire_gather(g, g % _NBUF)

            def body(i, carry):
                for b in range(_NBUF):
                    g = i + b
                    wait_gather(g, b)
                    fire_scatter(g, b)

                    @pl.when(g + _LOOK < nch)
                    def _():
                        fire_gather(g + _LOOK, (b + _LOOK) % _NBUF)

                    @pl.when(g >= _SCAT)
                    def _():
                        wait_scatter(g - _SCAT, (b - _SCAT) % _NBUF)
                return carry

            lax.fori_loop(0, nch // _NBUF, lambda i, cr: body(i * _NBUF, cr),
                          0)
            for g in range(nch - _SCAT, nch):
                wait_scatter(g, g % _NBUF)

        def writeout(phase):
            pltpu.sync_copy(acc.at[row_slice],
                            out_hbm.at[phase, c].at[row_slice])
            if rem:
                @pl.when(s == _TILES - 1)
                def _():
                    pltpu.sync_copy(acc.at[tail_slice],
                                    out_hbm.at[phase, c].at[tail_slice])

        for phase in (0, 1):
            zero_acc()
            plsc.subcore_barrier()
            accumulate(phase)
            plsc.subcore_barrier()
            writeout(phase)
            if phase == 0:
                plsc.subcore_barrier()

    return seg


def _segment_sum(x, src2, dst):
    n, d = x.shape
    dh = d // 2
    e = dst.size
    zrows = jnp.zeros(((n // _TILES) // 8 * 8, dh), jnp.float32)
    return _make_segment_sum(n, e, dh)(x.reshape(2 * n, dh), src2, dst, zrows)


def _prep_edges(idx):
    e = idx.shape[1]
    shp = (_CORES * _TILES, e // (_CORES * _TILES * _CHUNK), _CHUNK)
    src2 = (idx[0] * 2).reshape(shp)
    return jnp.stack([src2, src2 + 1]), idx[1].reshape(shp)


_BLK = 1000


def _agg_from_partials(agg_ref):
    return jnp.concatenate([agg_ref[0, 0] + agg_ref[0, 1],
                            agg_ref[1, 0] + agg_ref[1, 1]], axis=1)


def _mlp1_body(x_ref, agg_ref, w1a_ref, b1a_ref, w1b_ref, b1b_ref, w2a_ref,
               t_ref):
    xa = x_ref[...] + _agg_from_partials(agg_ref)
    g = jnp.maximum(
        jnp.dot(xa, w1a_ref[...], preferred_element_type=jnp.float32)
        + b1a_ref[...], 0.0)
    h = jnp.maximum(
        jnp.dot(g, w1b_ref[...], preferred_element_type=jnp.float32)
        + b1b_ref[...], 0.0)
    t_ref[...] = jnp.dot(h, w2a_ref[...], preferred_element_type=jnp.float32)


def _mlp1(x, agg, p):
    n, d_in = x.shape
    hid = p["W1a"].shape[1]
    lat = p["W2a"].shape[1]
    grid = (n // _BLK,)
    full = lambda shape: pl.BlockSpec(shape, lambda i: (0,) * len(shape))
    return pl.pallas_call(
        _mlp1_body,
        grid=grid,
        in_specs=[
            pl.BlockSpec((_BLK, d_in), lambda i: (i, 0)),
            pl.BlockSpec((2, _CORES, _BLK, d_in // 2),
                         lambda i: (0, 0, i, 0)),
            full((d_in, hid)), full((1, hid)),
            full((hid, hid)), full((1, hid)),
            full((hid, lat)),
        ],
        out_specs=pl.BlockSpec((_BLK, lat), lambda i: (i, 0)),
        out_shape=jax.ShapeDtypeStruct((n, lat), jnp.float32),
    )(x, agg, p["W1a"], p["b1a"].reshape(1, -1), p["W1b"],
      p["b1b"].reshape(1, -1), p["W2a"])


def _mlp2_body(t_ref, aggt_ref, b2a_ref, w2b_ref, b2b_ref, wp_ref, bp_ref,
               o_ref):
    z = jnp.maximum(t_ref[...] + _agg_from_partials(aggt_ref) + b2a_ref[...],
                    0.0)
    h2 = jnp.dot(z, w2b_ref[...], preferred_element_type=jnp.float32) \
        + b2b_ref[...]
    o_ref[...] = jnp.dot(h2, wp_ref[...], preferred_element_type=jnp.float32) \
        + bp_ref[...]


def _mlp2(t, aggt, p):
    n, lat = t.shape
    grid = (n // _BLK,)
    full = lambda shape: pl.BlockSpec(shape, lambda i: (0,) * len(shape))
    return pl.pallas_call(
        _mlp2_body,
        grid=grid,
        in_specs=[
            pl.BlockSpec((_BLK, lat), lambda i: (i, 0)),
            pl.BlockSpec((2, _CORES, _BLK, lat // 2),
                         lambda i: (0, 0, i, 0)),
            full((1, lat)),
            full((lat, lat)), full((1, lat)),
            full((lat, lat)), full((1, lat)),
        ],
        out_specs=pl.BlockSpec((_BLK, lat), lambda i: (i, 0)),
        out_shape=jax.ShapeDtypeStruct((n, lat), jnp.float32),
    )(t, aggt, p["b2a"].reshape(1, -1), p["W2b"],
      p["b2b"].reshape(1, -1), p["Wp"], p["bp"].reshape(1, -1))


def kernel(emg_x, eeg_x, emg_edge_index, eeg_edge_index, emg_params,
           eeg_params):
    src2_emg, dst_emg = _prep_edges(emg_edge_index)
    src2_eeg, dst_eeg = _prep_edges(eeg_edge_index)
    agg_emg = _segment_sum(emg_x, src2_emg, dst_emg)
    agg_eeg = _segment_sum(eeg_x, src2_eeg, dst_eeg)
    t_emg = _mlp1(emg_x, agg_emg, emg_params)
    aggt_emg = _segment_sum(t_emg, src2_emg, dst_emg)
    t_eeg = _mlp1(eeg_x, agg_eeg, eeg_params)
    aggt_eeg = _segment_sum(t_eeg, src2_eeg, dst_eeg)
    o_emg = _mlp2(t_emg, aggt_emg, emg_params)
    o_eeg = _mlp2(t_eeg, aggt_eeg, eeg_params)
    return jnp.concatenate([o_emg, o_eeg], axis=0)
